# baseline (speedup 1.0000x reference)
import functools

import jax
import jax.numpy as jnp
from jax import lax
from jax.experimental import pallas as pl
from jax.experimental.pallas import tpu as pltpu
from jax.experimental.pallas import tpu_sc as plsc

NC = 2
NS = 16
NW = NC * NS
L = 16

B, H, W = 8, 384, 384
N = B * H * W
PIX_PER_W = N // NW
CHUNK = 9216
NCHUNK = PIX_PER_W // CHUNK
STEPS = CHUNK // L


def _sc_partials_kernel(x_hbm, part_hbm,
                        gb0, gb1, hb0, hb1, cb0, cb1, ab0, ab1, vec16,
                        sem0, sem1):
    wid = lax.axis_index("s") * NC + lax.axis_index("c")
    base_pix = wid * PIX_PER_W

    gbufs = (gb0, gb1)
    hbufs = (hb0, hb1)
    cbufs = (cb0, cb1)
    abufs = (ab0, ab1)
    sems = (sem0, sem1)

    def issue(c, b):
        pix0 = base_pix + c * CHUNK
        return (
            pltpu.async_copy(x_hbm.at[pl.ds(pix0, CHUNK)], gbufs[b], sems[b]),
            pltpu.async_copy(x_hbm.at[pl.ds(N + pix0, CHUNK)], hbufs[b],
                             sems[b]),
            pltpu.async_copy(x_hbm.at[pl.ds(2 * N + pix0, CHUNK)], cbufs[b],
                             sems[b]),
            pltpu.async_copy(x_hbm.at[pl.ds(3 * N + pix0, CHUNK)], abufs[b],
                             sems[b]),
        )

    iota = lax.iota(jnp.int32, L)
    zero = jnp.zeros((L,), jnp.float32)
    one = jnp.ones((L,), jnp.float32)

    def chunk_body(gb, hb, cb, ab, accs):
        def step(i, accs):
            (psc, nsc, cpc, cnc, psa, nsa, cpa, cna) = accs
            g_c = gb[pl.ds(i * L, L)]
            g_a = hb[pl.ds(i * L, L)]
            t_c = cb[pl.ds(i * L, L)]
            t_a = ab[pl.ds(i * L, L)]

            d_c = g_c - t_c
            l_c = d_c * d_c
            pos_c = t_c >= 0.1
            neg_c = t_c <= 0.0
            psc = psc + jnp.where(pos_c, l_c, zero)
            nsc = nsc + jnp.where(neg_c, l_c, zero)
            cpc = cpc + jnp.where(pos_c, one, zero)
            cnc = cnc + jnp.where(neg_c, one, zero)

            d_a = g_a - t_a
            l_a = d_a * d_a
            pos_a = t_a >= 0.1
            neg_a = t_a <= 0.0
            psa = psa + jnp.where(pos_a, l_a, zero)
            nsa = nsa + jnp.where(neg_a, l_a, zero)
            cpa = cpa + jnp.where(pos_a, one, zero)
            cna = cna + jnp.where(neg_a, one, zero)
            return (psc, nsc, cpc, cnc, psa, nsa, cpa, cna)

        return lax.fori_loop(0, STEPS, step, accs)

    accs = (zero,) * 8
    inflight = issue(0, 0)
    for c in range(NCHUNK):
        b = c % 2
        for cp in inflight:
            cp.wait()
        if c + 1 < NCHUNK:
            inflight = issue(c + 1, 1 - b)
        accs = chunk_body(gbufs[b], hbufs[b], cbufs[b], abufs[b], accs)

    packed = zero
    for j, acc in enumerate(accs):
        s = jnp.sum(acc)
        packed = jnp.where(iota == j, s, packed)
    vec16[...] = packed
    pltpu.sync_copy(vec16, part_hbm.at[wid])


def _sc_finalize_kernel(part_hbm, stats_hbm, buf, vec16):
    wid = lax.axis_index("s") * NC + lax.axis_index("c")

    @pl.when(wid == 0)
    def _():
        pltpu.sync_copy(part_hbm, buf)
        tot = jnp.zeros((L,), jnp.float32)
        for i in range(NW):
            tot = tot + buf[i]
        cpc, cnc = tot[2], tot[3]
        cpa, cna = tot[6], tot[7]

        iota = lax.iota(jnp.int32, L)
        kcap_c = jnp.maximum(jnp.float32(1000.0), 3.0 * cpc)
        kcap_a = jnp.maximum(jnp.float32(1000.0), 3.0 * cpa)
        flag_c = jnp.where(kcap_c >= cnc, jnp.float32(1.0), jnp.float32(0.0))
        flag_a = jnp.where(kcap_a >= cna, jnp.float32(1.0), jnp.float32(0.0))

        outv = jnp.zeros((L,), jnp.float32)
        outv = jnp.where(iota == 0, flag_c, outv)
        outv = jnp.where(iota == 1, flag_a, outv)
        for j in range(8):
            outv = jnp.where(iota == j + 2, tot[j], outv)
        vec16[...] = outv
        pltpu.sync_copy(vec16, stats_hbm)


def _make_sc_partials():
    mesh = plsc.VectorSubcoreMesh(core_axis_name="c", subcore_axis_name="s")
    return pl.kernel(
        _sc_partials_kernel,
        out_type=jax.ShapeDtypeStruct((NW, L), jnp.float32),
        mesh=mesh,
        compiler_params=pltpu.CompilerParams(needs_layout_passes=False),
        scratch_types=(
            [pltpu.VMEM((CHUNK,), jnp.float32) for _ in range(8)]
            + [pltpu.VMEM((L,), jnp.float32),
               pltpu.SemaphoreType.DMA,
               pltpu.SemaphoreType.DMA]
        ),
    )


def _make_sc_finalize():
    mesh = plsc.VectorSubcoreMesh(core_axis_name="c", subcore_axis_name="s")
    return pl.kernel(
        _sc_finalize_kernel,
        out_type=jax.ShapeDtypeStruct((L,), jnp.float32),
        mesh=mesh,
        compiler_params=pltpu.CompilerParams(needs_layout_passes=False),
        scratch_types=[
            pltpu.VMEM((NW, L), jnp.float32),
            pltpu.VMEM((L,), jnp.float32),
        ],
    )


_R, _C = 1152, 1024


def _topk_sum_kernel(pred_ref, targ_ref, k_ref, out_ref, enc_ref):
    p = pred_ref[...]
    t = targ_ref[...]
    d = p - t
    l = d * d
    neg = t <= 0.0
    bits = lax.bitcast_convert_type(l, jnp.int32)
    enc = jnp.where(neg, bits + 1, 0)
    enc_ref[...] = enc
    k = k_ref[0]

    def body(i, thr):
        cand = thr | (jnp.int32(1) << (30 - i))
        cnt = jnp.sum((enc_ref[...] >= cand).astype(jnp.int32))
        return jnp.where(cnt >= k, cand, thr)

    thr = lax.fori_loop(0, 31, body, jnp.int32(0))
    enc2 = enc_ref[...]
    above = enc2 >= thr + 1
    cnt_gt = jnp.sum(above.astype(jnp.int32))
    vals = lax.bitcast_convert_type(enc2 - 1, jnp.float32)
    sum_gt = jnp.sum(jnp.where(above, vals, 0.0))
    tval = lax.bitcast_convert_type(thr - 1, jnp.float32)
    sel = sum_gt + (k - cnt_gt).astype(jnp.float32) * tval
    out_ref[0] = jnp.where(k > 0, sel, jnp.float32(0.0))


def _topk_sum(pred2d, targ2d, k_i32):
    call = pl.pallas_call(
        _topk_sum_kernel,
        out_shape=jax.ShapeDtypeStruct((1,), jnp.float32),
        in_specs=[
            pl.BlockSpec(memory_space=pltpu.VMEM),
            pl.BlockSpec(memory_space=pltpu.VMEM),
            pl.BlockSpec(memory_space=pltpu.SMEM),
        ],
        out_specs=pl.BlockSpec(memory_space=pltpu.SMEM),
        scratch_shapes=[pltpu.VMEM((_R, _C), jnp.int32)],
    )
    return call(pred2d, targ2d, k_i32.reshape(1))[0]


def kernel(output, character_map, affinity_map):
    ch = output[..., 0].reshape(-1)
    af = output[..., 1].reshape(-1)
    cm = character_map.reshape(-1)
    am = affinity_map.reshape(-1)
    x = jnp.concatenate([ch, af, cm, am])

    partials = _make_sc_partials()(x)
    stats = _make_sc_finalize()(partials)

    flag_c, flag_a = stats[0], stats[1]
    psc, nsc, npc, nnc = stats[2], stats[3], stats[4], stats[5]
    psa, nsa, npa, nna = stats[6], stats[7], stats[8], stats[9]

    def case_a(_):
        loss_c = (psc + nsc) / (npc + nnc)
        loss_a = (psa + nsa) / (npa + nna)
        return loss_c * 2 + loss_a

    def case_b(_):
        kc = jnp.minimum(jnp.maximum(jnp.float32(1000.0), 3.0 * npc), nnc)
        ka = jnp.minimum(jnp.maximum(jnp.float32(1000.0), 3.0 * npa), nna)
        pred_c = output[..., 0].reshape(_R, _C)
        pred_a = output[..., 1].reshape(_R, _C)
        tc = character_map.reshape(_R, _C)
        ta = affinity_map.reshape(_R, _C)
        sel_c = _topk_sum(pred_c, tc, kc.astype(jnp.int32))
        sel_a = _topk_sum(pred_a, ta, ka.astype(jnp.int32))
        loss_c = (psc + sel_c) / (npc + kc)
        loss_a = (psa + sel_a) / (npa + ka)
        return loss_c * 2 + loss_a

    both_a = jnp.logical_and(flag_c > 0.5, flag_a > 0.5)
    return lax.cond(both_a, case_a, case_b, operand=None)


# speedup vs baseline: 1.4148x; 1.4148x over previous
import functools

import jax
import jax.numpy as jnp
from jax import lax
from jax.experimental import pallas as pl
from jax.experimental.pallas import tpu as pltpu
from jax.experimental.pallas import tpu_sc as plsc

NC = 2
NS = 16
NW = NC * NS
L = 16

B, H, W = 8, 384, 384
N = B * H * W
PIX_PER_W = N // NW
CHUNK = 9216
NCHUNK = PIX_PER_W // CHUNK
STEPS = CHUNK // L


def _sc_partials_kernel(ch_hbm, af_hbm, cm_hbm, am_hbm, part_hbm,
                        gb0, gb1, hb0, hb1, cb0, cb1, ab0, ab1, vec16,
                        sem0, sem1):
    wid = lax.axis_index("s") * NC + lax.axis_index("c")
    base_pix = wid * PIX_PER_W

    gbufs = (gb0, gb1)
    hbufs = (hb0, hb1)
    cbufs = (cb0, cb1)
    abufs = (ab0, ab1)
    sems = (sem0, sem1)

    def issue(c, b):
        pix0 = base_pix + c * CHUNK
        return (
            pltpu.async_copy(ch_hbm.at[pl.ds(pix0, CHUNK)], gbufs[b], sems[b]),
            pltpu.async_copy(af_hbm.at[pl.ds(pix0, CHUNK)], hbufs[b], sems[b]),
            pltpu.async_copy(cm_hbm.at[pl.ds(pix0, CHUNK)], cbufs[b], sems[b]),
            pltpu.async_copy(am_hbm.at[pl.ds(pix0, CHUNK)], abufs[b], sems[b]),
        )

    iota = lax.iota(jnp.int32, L)
    zero = jnp.zeros((L,), jnp.float32)
    zeroi = jnp.zeros((L,), jnp.int32)

    def chunk_body(gb, hb, cb, ab, accs):
        def step(i, accs):
            (psc, nsc, cpc, cnc, psa, nsa, cpa, cna) = accs
            g_c = gb[pl.ds(i * L, L)]
            g_a = hb[pl.ds(i * L, L)]
            t_c = cb[pl.ds(i * L, L)]
            t_a = ab[pl.ds(i * L, L)]

            d_c = g_c - t_c
            l_c = d_c * d_c
            pos_c = t_c >= 0.1
            neg_c = t_c <= 0.0
            psc = psc + jnp.where(pos_c, l_c, zero)
            nsc = nsc + jnp.where(neg_c, l_c, zero)
            cpc = cpc + plsc.all_reduce_population_count(pos_c)
            cnc = cnc + plsc.all_reduce_population_count(neg_c)

            d_a = g_a - t_a
            l_a = d_a * d_a
            pos_a = t_a >= 0.1
            neg_a = t_a <= 0.0
            psa = psa + jnp.where(pos_a, l_a, zero)
            nsa = nsa + jnp.where(neg_a, l_a, zero)
            cpa = cpa + plsc.all_reduce_population_count(pos_a)
            cna = cna + plsc.all_reduce_population_count(neg_a)
            return (psc, nsc, cpc, cnc, psa, nsa, cpa, cna)

        return lax.fori_loop(0, STEPS, step, accs)

    accs = (zero, zero, zeroi, zeroi, zero, zero, zeroi, zeroi)
    inflight = issue(0, 0)
    for c in range(NCHUNK):
        b = c % 2
        for cp in inflight:
            cp.wait()
        if c + 1 < NCHUNK:
            inflight = issue(c + 1, 1 - b)
        accs = chunk_body(gbufs[b], hbufs[b], cbufs[b], abufs[b], accs)

    packed = zero
    for j, acc in enumerate(accs):
        if acc.dtype == jnp.int32:
            s = acc[0].astype(jnp.float32)
        else:
            s = jnp.sum(acc)
        packed = jnp.where(iota == j, s, packed)
    vec16[...] = packed
    pltpu.sync_copy(vec16, part_hbm.at[wid])


def _sc_finalize_kernel(part_hbm, stats_hbm, buf, vec16):
    wid = lax.axis_index("s") * NC + lax.axis_index("c")

    @pl.when(wid == 0)
    def _():
        pltpu.sync_copy(part_hbm, buf)
        tot = jnp.zeros((L,), jnp.float32)
        for i in range(NW):
            tot = tot + buf[i]
        cpc, cnc = tot[2], tot[3]
        cpa, cna = tot[6], tot[7]

        iota = lax.iota(jnp.int32, L)
        kcap_c = jnp.maximum(jnp.float32(1000.0), 3.0 * cpc)
        kcap_a = jnp.maximum(jnp.float32(1000.0), 3.0 * cpa)
        flag_c = jnp.where(kcap_c >= cnc, jnp.float32(1.0), jnp.float32(0.0))
        flag_a = jnp.where(kcap_a >= cna, jnp.float32(1.0), jnp.float32(0.0))

        outv = jnp.zeros((L,), jnp.float32)
        outv = jnp.where(iota == 0, flag_c, outv)
        outv = jnp.where(iota == 1, flag_a, outv)
        for j in range(8):
            outv = jnp.where(iota == j + 2, tot[j], outv)
        vec16[...] = outv
        pltpu.sync_copy(vec16, stats_hbm)


def _make_sc_partials():
    mesh = plsc.VectorSubcoreMesh(core_axis_name="c", subcore_axis_name="s")
    return pl.kernel(
        _sc_partials_kernel,
        out_type=jax.ShapeDtypeStruct((NW, L), jnp.float32),
        mesh=mesh,
        compiler_params=pltpu.CompilerParams(needs_layout_passes=False),
        scratch_types=(
            [pltpu.VMEM((CHUNK,), jnp.float32) for _ in range(8)]
            + [pltpu.VMEM((L,), jnp.float32),
               pltpu.SemaphoreType.DMA,
               pltpu.SemaphoreType.DMA]
        ),
    )


def _make_sc_finalize():
    mesh = plsc.VectorSubcoreMesh(core_axis_name="c", subcore_axis_name="s")
    return pl.kernel(
        _sc_finalize_kernel,
        out_type=jax.ShapeDtypeStruct((L,), jnp.float32),
        mesh=mesh,
        compiler_params=pltpu.CompilerParams(needs_layout_passes=False),
        scratch_types=[
            pltpu.VMEM((NW, L), jnp.float32),
            pltpu.VMEM((L,), jnp.float32),
        ],
    )


_R, _C = 1152, 1024


def _topk_sum_kernel(pred_ref, targ_ref, k_ref, out_ref, enc_ref):
    p = pred_ref[...]
    t = targ_ref[...]
    d = p - t
    l = d * d
    neg = t <= 0.0
    bits = lax.bitcast_convert_type(l, jnp.int32)
    enc = jnp.where(neg, bits + 1, 0)
    enc_ref[...] = enc
    k = k_ref[0]

    def body(i, thr):
        cand = thr | (jnp.int32(1) << (30 - i))
        cnt = jnp.sum((enc_ref[...] >= cand).astype(jnp.int32))
        return jnp.where(cnt >= k, cand, thr)

    thr = lax.fori_loop(0, 31, body, jnp.int32(0))
    enc2 = enc_ref[...]
    above = enc2 >= thr + 1
    cnt_gt = jnp.sum(above.astype(jnp.int32))
    vals = lax.bitcast_convert_type(enc2 - 1, jnp.float32)
    sum_gt = jnp.sum(jnp.where(above, vals, 0.0))
    tval = lax.bitcast_convert_type(thr - 1, jnp.float32)
    sel = sum_gt + (k - cnt_gt).astype(jnp.float32) * tval
    out_ref[0] = jnp.where(k > 0, sel, jnp.float32(0.0))


def _topk_sum(pred2d, targ2d, k_i32):
    call = pl.pallas_call(
        _topk_sum_kernel,
        out_shape=jax.ShapeDtypeStruct((1,), jnp.float32),
        in_specs=[
            pl.BlockSpec(memory_space=pltpu.VMEM),
            pl.BlockSpec(memory_space=pltpu.VMEM),
            pl.BlockSpec(memory_space=pltpu.SMEM),
        ],
        out_specs=pl.BlockSpec(memory_space=pltpu.SMEM),
        scratch_shapes=[pltpu.VMEM((_R, _C), jnp.int32)],
    )
    return call(pred2d, targ2d, k_i32.reshape(1))[0]


def kernel(output, character_map, affinity_map):
    ch = output[..., 0].reshape(-1)
    af = output[..., 1].reshape(-1)
    cm = character_map.reshape(-1)
    am = affinity_map.reshape(-1)

    partials = _make_sc_partials()(ch, af, cm, am)
    stats = _make_sc_finalize()(partials)

    flag_c, flag_a = stats[0], stats[1]
    psc, nsc, npc, nnc = stats[2], stats[3], stats[4], stats[5]
    psa, nsa, npa, nna = stats[6], stats[7], stats[8], stats[9]

    def case_a(_):
        loss_c = (psc + nsc) / (npc + nnc)
        loss_a = (psa + nsa) / (npa + nna)
        return loss_c * 2 + loss_a

    def case_b(_):
        kc = jnp.minimum(jnp.maximum(jnp.float32(1000.0), 3.0 * npc), nnc)
        ka = jnp.minimum(jnp.maximum(jnp.float32(1000.0), 3.0 * npa), nna)
        pred_c = output[..., 0].reshape(_R, _C)
        pred_a = output[..., 1].reshape(_R, _C)
        tc = character_map.reshape(_R, _C)
        ta = affinity_map.reshape(_R, _C)
        sel_c = _topk_sum(pred_c, tc, kc.astype(jnp.int32))
        sel_a = _topk_sum(pred_a, ta, ka.astype(jnp.int32))
        loss_c = (psc + sel_c) / (npc + kc)
        loss_a = (psa + sel_a) / (npa + ka)
        return loss_c * 2 + loss_a

    both_a = jnp.logical_and(flag_c > 0.5, flag_a > 0.5)
    return lax.cond(both_a, case_a, case_b, operand=None)
